# baseline (speedup 1.0000x reference)
import jax
import jax.numpy as jnp
from jax import lax
from jax.experimental import pallas as pl

B, T, C = 128, 32, 10000
ATT = 64
ROWS = B * T
RB = 128
NRB = ROWS // RB


def _fused_body(x_ref, w1_ref, b1_ref, w2_ref, tc_ref, msk_ref, out_ref):
    x = x_ref[...]
    s = jax.nn.sigmoid(x)
    acc = jnp.dot(s, w1_ref[...], preferred_element_type=jnp.float32)
    e = jnp.tanh(acc + b1_ref[...])
    logits = jnp.dot(e, w2_ref[...],
                     preferred_element_type=jnp.float32)
    l = jnp.where(msk_ref[...] > 0, logits, -30.0)
    p = jnp.exp(l)
    ri = lax.broadcasted_iota(jnp.int32, (RB, RB), 0) // T
    ci = lax.broadcasted_iota(jnp.int32, (RB, RB), 1) // T
    g = (ri == ci).astype(jnp.float32)
    gsum = jnp.dot(g, p, preferred_element_type=jnp.float32)
    score = p / gsum
    code_ids = lax.broadcasted_iota(jnp.int32, (RB, C), 1)
    hit = code_ids == tc_ref[...]
    xg = jnp.sum(jnp.where(hit, x, 0.0), axis=1, keepdims=True)
    vals = jax.nn.sigmoid(xg + score)
    out_ref[...] = jnp.where(hit, vals, s)


def kernel(x, lens, target_codes, W1, b1, w2):
    x2 = x.reshape(ROWS, C)
    tc2 = target_codes.reshape(ROWS, 1)
    t_of_row = jnp.tile(jnp.arange(T, dtype=jnp.int32), B)
    msk = (t_of_row < jnp.repeat(lens, T)).astype(jnp.int32).reshape(ROWS, 1)
    out2 = pl.pallas_call(
        _fused_body,
        grid=(NRB,),
        in_specs=[
            pl.BlockSpec((RB, C), lambda r: (r, 0)),
            pl.BlockSpec((C, ATT), lambda r: (0, 0)),
            pl.BlockSpec((1, ATT), lambda r: (0, 0)),
            pl.BlockSpec((ATT, 1), lambda r: (0, 0)),
            pl.BlockSpec((RB, 1), lambda r: (r, 0)),
            pl.BlockSpec((RB, 1), lambda r: (r, 0)),
        ],
        out_specs=pl.BlockSpec((RB, C), lambda r: (r, 0)),
        out_shape=jax.ShapeDtypeStruct((ROWS, C), jnp.float32),
    )(x2, W1, b1.reshape(1, ATT), w2.reshape(ATT, 1), tc2, msk)
    return out2.reshape(B, T, C)


# speedup vs baseline: 1.6560x; 1.6560x over previous
import functools

import jax
import jax.numpy as jnp
from jax import lax
from jax.experimental import pallas as pl
from jax.experimental.pallas import tpu as pltpu
from jax.experimental.pallas import tpu_sc as plsc

B, T, C = 128, 32, 10000
ATT = 64
ROWS = B * T
CB = 2000
NCB = C // CB


def _stream_body(xp_ref, w1_ref, b1_ref, w2_ref, tcp_ref, lens_ref,
                 y_ref, vals_ref, acc_ref, xg_ref, logit_ref):
    c = pl.program_id(0)
    t = pl.program_id(1)
    x = xp_ref[0]
    s = jax.nn.sigmoid(x)
    y_ref[...] = s
    partial = lax.dot_general(w1_ref[...], s, (((0,), (0,)), ((), ())),
                              preferred_element_type=jnp.float32)
    code_ids = c * CB + lax.broadcasted_iota(jnp.int32, (CB, B), 0)
    hit = code_ids == tcp_ref[0]
    xgp = jnp.sum(jnp.where(hit, x, 0.0), axis=0, keepdims=True)

    @pl.when(c == 0)
    def _():
        acc_ref[t] = partial
        xg_ref[pl.ds(t, 1), :] = xgp

    @pl.when(c > 0)
    def _():
        acc_ref[t] += partial
        xg_ref[pl.ds(t, 1), :] += xgp

    @pl.when(c == NCB - 1)
    def _():
        e = jnp.tanh(acc_ref[t] + b1_ref[...])
        lg = lax.dot_general(w2_ref[...], e, (((0,), (0,)), ((), ())),
                             preferred_element_type=jnp.float32)
        logit_ref[pl.ds(t, 1), :] = lg

    @pl.when((c == NCB - 1) & (t == T - 1))
    def _():
        t_ids = lax.broadcasted_iota(jnp.int32, (T, B), 0)
        mask = t_ids < lens_ref[...]
        l = jnp.where(mask, logit_ref[...], -30.0)
        p = jnp.exp(l)
        score = p / jnp.sum(p, axis=0, keepdims=True)
        vals_ref[...] = jax.nn.sigmoid(xg_ref[...] + score)


def _make_scatter():
    info = plsc.get_sparse_core_info()
    nw = info.num_cores * info.num_subcores
    per_w = ROWS // nw

    mesh = plsc.VectorSubcoreMesh(core_axis_name="c", subcore_axis_name="s")

    @functools.partial(
        pl.kernel, mesh=mesh, out_type=(),
        scratch_types=[
            pltpu.VMEM((per_w,), jnp.int32),
            pltpu.VMEM((per_w,), jnp.float32),
            pltpu.SemaphoreType.DMA,
        ],
    )
    def scatter(idx_hbm, vals_hbm, y_ref, idx_v, vals_v, sem):
        wid = lax.axis_index("s") * info.num_cores + lax.axis_index("c")
        base = wid * per_w
        pltpu.sync_copy(idx_hbm.at[pl.ds(base, per_w)], idx_v)
        pltpu.sync_copy(vals_hbm.at[pl.ds(base, per_w)], vals_v)
        pltpu.async_copy(vals_v, y_ref.at[idx_v], sem).wait()

    return scatter


_scatter = None


def kernel(x, lens, target_codes, W1, b1, w2):
    global _scatter
    if _scatter is None:
        _scatter = _make_scatter()

    xp = jnp.transpose(x, (1, 2, 0))
    tcp = jnp.transpose(target_codes, (1, 0)).reshape(T, 1, B)
    lens2 = lens.reshape(1, B)

    y2d, vals = pl.pallas_call(
        _stream_body,
        grid=(NCB, T),
        in_specs=[
            pl.BlockSpec((1, CB, B), lambda c, t: (t, c, 0)),
            pl.BlockSpec((CB, ATT), lambda c, t: (c, 0)),
            pl.BlockSpec((ATT, 1), lambda c, t: (0, 0)),
            pl.BlockSpec((ATT, 1), lambda c, t: (0, 0)),
            pl.BlockSpec((1, 1, B), lambda c, t: (t, 0, 0)),
            pl.BlockSpec((1, B), lambda c, t: (0, 0)),
        ],
        out_specs=[
            pl.BlockSpec((CB, B), lambda c, t: (t * NCB + c, 0)),
            pl.BlockSpec((T, B), lambda c, t: (0, 0)),
        ],
        out_shape=[
            jax.ShapeDtypeStruct((T * C, B), jnp.float32),
            jax.ShapeDtypeStruct((T, B), jnp.float32),
        ],
        scratch_shapes=[
            pltpu.VMEM((T, ATT, B), jnp.float32),
            pltpu.VMEM((T, B), jnp.float32),
            pltpu.VMEM((T, B), jnp.float32),
        ],
    )(xp, W1, b1.reshape(ATT, 1), w2.reshape(ATT, 1), tcp, lens2)

    tt = jnp.arange(T, dtype=jnp.int32).reshape(T, 1)
    bb = jnp.arange(B, dtype=jnp.int32).reshape(1, B)
    idx = ((tt * C + jnp.transpose(target_codes, (1, 0))) * B
           + bb).reshape(ROWS)

    y_ref = jax.new_ref(y2d.reshape(T * C * B))
    _scatter(idx, vals.reshape(ROWS), y_ref)
    yf = jax.freeze(y_ref)
    return jnp.transpose(yf.reshape(T, C, B), (2, 0, 1))


# speedup vs baseline: 2.5607x; 1.5464x over previous
import functools

import jax
import jax.numpy as jnp
from jax import lax
from jax.experimental import pallas as pl
from jax.experimental.pallas import tpu as pltpu
from jax.experimental.pallas import tpu_sc as plsc

B, T, C = 128, 32, 10000
ATT = 64
ROWS = B * T
CB = 10000
NCB = C // CB


def _stream_body(xp_ref, w1_ref, b1_ref, w2_ref, tcp_ref, lens_ref,
                 y_ref, vals_ref, acc_ref, xg_ref, logit_ref):
    c = pl.program_id(0)
    t = pl.program_id(1)
    x = xp_ref[0]
    s = jax.nn.sigmoid(x)
    y_ref[...] = s
    partial = lax.dot_general(w1_ref[...], s, (((0,), (0,)), ((), ())),
                              preferred_element_type=jnp.float32)
    code_ids = c * CB + lax.broadcasted_iota(jnp.int32, (CB, B), 0)
    hit = code_ids == tcp_ref[0]
    xgp = jnp.sum(jnp.where(hit, x, 0.0), axis=0, keepdims=True)

    @pl.when(c == 0)
    def _():
        acc_ref[t] = partial
        xg_ref[pl.ds(t, 1), :] = xgp

    @pl.when(c > 0)
    def _():
        acc_ref[t] += partial
        xg_ref[pl.ds(t, 1), :] += xgp

    @pl.when(c == NCB - 1)
    def _():
        e = jnp.tanh(acc_ref[t] + b1_ref[...])
        lg = lax.dot_general(w2_ref[...], e, (((0,), (0,)), ((), ())),
                             preferred_element_type=jnp.float32)
        logit_ref[pl.ds(t, 1), :] = lg

    @pl.when((c == NCB - 1) & (t == T - 1))
    def _():
        t_ids = lax.broadcasted_iota(jnp.int32, (T, B), 0)
        mask = t_ids < lens_ref[...]
        l = jnp.where(mask, logit_ref[...], -30.0)
        p = jnp.exp(l)
        score = p / jnp.sum(p, axis=0, keepdims=True)
        vals_ref[...] = jax.nn.sigmoid(xg_ref[...] + score)


def _make_scatter():
    info = plsc.get_sparse_core_info()
    nw = info.num_cores * info.num_subcores
    per_w = ROWS // nw

    mesh = plsc.VectorSubcoreMesh(core_axis_name="c", subcore_axis_name="s")

    @functools.partial(
        pl.kernel, mesh=mesh, out_type=(),
        scratch_types=[
            pltpu.VMEM((per_w,), jnp.int32),
            pltpu.VMEM((per_w,), jnp.float32),
            pltpu.SemaphoreType.DMA,
        ],
    )
    def scatter(idx_hbm, vals_hbm, y_ref, idx_v, vals_v, sem):
        wid = lax.axis_index("s") * info.num_cores + lax.axis_index("c")
        base = wid * per_w
        pltpu.sync_copy(idx_hbm.at[pl.ds(base, per_w)], idx_v)
        pltpu.sync_copy(vals_hbm.at[pl.ds(base, per_w)], vals_v)
        pltpu.async_copy(vals_v, y_ref.at[idx_v], sem).wait()

    return scatter


_scatter = None


def kernel(x, lens, target_codes, W1, b1, w2):
    global _scatter
    if _scatter is None:
        _scatter = _make_scatter()

    xp = jnp.transpose(x, (1, 2, 0))
    tcp = jnp.transpose(target_codes, (1, 0)).reshape(T, 1, B)
    lens2 = lens.reshape(1, B)

    y2d, vals = pl.pallas_call(
        _stream_body,
        grid=(NCB, T),
        in_specs=[
            pl.BlockSpec((1, CB, B), lambda c, t: (t, c, 0)),
            pl.BlockSpec((CB, ATT), lambda c, t: (c, 0)),
            pl.BlockSpec((ATT, 1), lambda c, t: (0, 0)),
            pl.BlockSpec((ATT, 1), lambda c, t: (0, 0)),
            pl.BlockSpec((1, 1, B), lambda c, t: (t, 0, 0)),
            pl.BlockSpec((1, B), lambda c, t: (0, 0)),
        ],
        out_specs=[
            pl.BlockSpec((CB, B), lambda c, t: (t * NCB + c, 0)),
            pl.BlockSpec((T, B), lambda c, t: (0, 0)),
        ],
        out_shape=[
            jax.ShapeDtypeStruct((T * C, B), jnp.float32),
            jax.ShapeDtypeStruct((T, B), jnp.float32),
        ],
        scratch_shapes=[
            pltpu.VMEM((T, ATT, B), jnp.float32),
            pltpu.VMEM((T, B), jnp.float32),
            pltpu.VMEM((T, B), jnp.float32),
        ],
    )(xp, W1, b1.reshape(ATT, 1), w2.reshape(ATT, 1), tcp, lens2)

    tt = jnp.arange(T, dtype=jnp.int32).reshape(T, 1)
    bb = jnp.arange(B, dtype=jnp.int32).reshape(1, B)
    idx = ((tt * C + jnp.transpose(target_codes, (1, 0))) * B
           + bb).reshape(ROWS)

    y_ref = jax.new_ref(y2d.reshape(T * C * B))
    _scatter(idx, vals.reshape(ROWS), y_ref)
    yf = jax.freeze(y_ref)
    return jnp.transpose(yf.reshape(T, C, B), (2, 0, 1))
